# Initial kernel scaffold; baseline (speedup 1.0000x reference)
import jax
import jax.numpy as jnp
from jax.experimental import pallas as pl


def kernel(x, edge_index, Wl1, bl1, Wr1, Wl2, bl2, Wr2, Wo, bo):
    raise NotImplementedError("write your pallas kernel here")


# speedup vs baseline: 4.5460x; 4.5460x over previous
import functools

import jax
import jax.numpy as jnp
from jax import lax
from jax.experimental import pallas as pl
from jax.experimental.pallas import tpu as pltpu
from jax.experimental.pallas import tpu_sc as plsc

NC = 2
NS = 16
NW = NC * NS
CH = 64


def _pad_sizes(n, e, d):
    dp = d + 16
    n_pad = ((n + 8 * NW - 1) // (8 * NW)) * (8 * NW)
    e_pad = ((e + 2 * NW * CH - 1) // (2 * NW * CH)) * (2 * NW * CH)
    return dp, n_pad, e_pad


def _sc_aggregate(feats_pad, src2d, dst2d, zeros_blk, n_pad, dp, nchunk):
    rows_tile = n_pad // NS
    mesh = plsc.VectorSubcoreMesh(core_axis_name="c", subcore_axis_name="s")

    @functools.partial(
        pl.kernel,
        mesh=mesh,
        compiler_params=pltpu.CompilerParams(use_tc_tiling_on_sc=False),
        out_type=jax.ShapeDtypeStruct((NC, n_pad, dp), jnp.float32),
        scratch_types=[
            pltpu.VMEM((CH,), jnp.int32),
            pltpu.VMEM((CH,), jnp.int32),
            pltpu.VMEM((CH,), jnp.int32),
            pltpu.VMEM((CH,), jnp.int32),
            pltpu.VMEM((CH, dp), jnp.float32),
            pltpu.VMEM((CH, dp), jnp.float32),
            pltpu.VMEM_SHARED((n_pad, dp), jnp.float32),
            pltpu.SemaphoreType.DMA,
            pltpu.SemaphoreType.DMA,
        ],
    )
    def k(feats_hbm, src_hbm, dst_hbm, zeros_hbm, out_hbm,
          sidx0, sidx1, didx0, didx1, rows0, rows1, acc, sem0, sem1):
        c = lax.axis_index("c")
        s = lax.axis_index("s")
        wid = s * NC + c

        pltpu.sync_copy(zeros_hbm, acc.at[pl.ds(s * rows_tile, rows_tile)])
        plsc.subcore_barrier()

        def load_idx(ch, sidx, didx):
            pltpu.sync_copy(src_hbm.at[wid, ch], sidx)
            pltpu.sync_copy(dst_hbm.at[wid, ch], didx)

        def gather(sidx, buf, sem):
            return pltpu.make_async_copy(feats_hbm.at[sidx], buf, sem)

        load_idx(0, sidx0, didx0)
        gather(sidx0, rows0, sem0).start()

        def body(g, carry):
            ch0 = 2 * g
            load_idx(ch0 + 1, sidx1, didx1)
            gather(sidx1, rows1, sem1).start()
            gather(sidx0, rows0, sem0).wait()
            pltpu.sync_copy(rows0, acc.at[didx0], add=True)

            @pl.when(ch0 + 2 < nchunk)
            def _():
                load_idx(ch0 + 2, sidx0, didx0)
                gather(sidx0, rows0, sem0).start()

            gather(sidx1, rows1, sem1).wait()
            pltpu.sync_copy(rows1, acc.at[didx1], add=True)
            return carry

        lax.fori_loop(0, nchunk // 2, body, 0)
        plsc.subcore_barrier()

        pltpu.sync_copy(acc.at[pl.ds(s * rows_tile, rows_tile)],
                        out_hbm.at[c, pl.ds(s * rows_tile, rows_tile)])

    return k(feats_pad, src2d, dst2d, zeros_blk)


def _combine1_body(p_ref, x_ref, wl_ref, bl_ref, wr_ref, o_ref, *, d, dp):
    p = p_ref[0] + p_ref[1]
    cnt = jnp.maximum(p[:, d:d + 1], 1.0)
    mean = p[:, :d] / cnt
    y = jnp.dot(mean, wl_ref[...], preferred_element_type=jnp.float32)
    y += jnp.dot(x_ref[:, :d], wr_ref[...], preferred_element_type=jnp.float32)
    y += bl_ref[...]
    y = jnp.maximum(y, 0.0)
    o_ref[:, :d] = y
    blk = y.shape[0]
    o_ref[:, d:] = jnp.concatenate(
        [jnp.ones((blk, 1), jnp.float32),
         jnp.zeros((blk, dp - d - 1), jnp.float32)], axis=1)


def _combine2_body(p_ref, h_ref, wl_ref, bl_ref, wr_ref, wo_ref, bo_ref,
                   out_ref, h2_ref, *, d):
    p = p_ref[0] + p_ref[1]
    cnt = jnp.maximum(p[:, d:d + 1], 1.0)
    mean = p[:, :d] / cnt
    h2 = jnp.dot(mean, wl_ref[...], preferred_element_type=jnp.float32)
    h2 += jnp.dot(h_ref[:, :d], wr_ref[...], preferred_element_type=jnp.float32)
    h2 += bl_ref[...]
    h2_ref[...] = h2
    out = jnp.dot(h2, wo_ref[...], preferred_element_type=jnp.float32)
    out_ref[...] = out + bo_ref[...]


def kernel(x, edge_index, Wl1, bl1, Wr1, Wl2, bl2, Wr2, Wo, bo):
    n, d = x.shape
    e = edge_index.shape[1]
    dp, n_pad, e_pad = _pad_sizes(n, e, d)
    nchunk = e_pad // NW // CH
    rows_tile = n_pad // NS
    blk = 512
    grid = (n_pad // blk,)

    src = jnp.concatenate(
        [edge_index[0], jnp.full((e_pad - e,), n, jnp.int32)]).reshape(NW, nchunk, CH)
    dst = jnp.concatenate(
        [edge_index[1], jnp.full((e_pad - e,), n, jnp.int32)]).reshape(NW, nchunk, CH)
    x_pad = jnp.zeros((n_pad, dp), jnp.float32)
    x_pad = x_pad.at[:n, :d].set(x).at[:n, d].set(1.0)
    zeros_blk = jnp.zeros((rows_tile, dp), jnp.float32)
    wl1t, wr1t = Wl1.T, Wr1.T
    wl2t, wr2t, wot = Wl2.T, Wr2.T, Wo.T
    bl1r, bl2r, bor = bl1.reshape(1, d), bl2.reshape(1, d), bo.reshape(1, d)

    wspec = pl.BlockSpec((d, d), lambda i: (0, 0))
    bspec = pl.BlockSpec((1, d), lambda i: (0, 0))
    pspec = pl.BlockSpec((NC, blk, dp), lambda i: (0, i, 0))
    fspec = pl.BlockSpec((blk, dp), lambda i: (i, 0))
    ospec = pl.BlockSpec((blk, d), lambda i: (i, 0))

    part1 = _sc_aggregate(x_pad, src, dst, zeros_blk, n_pad, dp, nchunk)
    h_pad = pl.pallas_call(
        functools.partial(_combine1_body, d=d, dp=dp),
        grid=grid,
        in_specs=[pspec, fspec, wspec, bspec, wspec],
        out_specs=fspec,
        out_shape=jax.ShapeDtypeStruct((n_pad, dp), jnp.float32),
    )(part1, x_pad, wl1t, bl1r, wr1t)

    part2 = _sc_aggregate(h_pad, src, dst, zeros_blk, n_pad, dp, nchunk)
    out_full, h2_full = pl.pallas_call(
        functools.partial(_combine2_body, d=d),
        grid=grid,
        in_specs=[pspec, fspec, wspec, bspec, wspec, wspec, bspec],
        out_specs=[ospec, ospec],
        out_shape=[jax.ShapeDtypeStruct((n_pad, d), jnp.float32),
                   jax.ShapeDtypeStruct((n_pad, d), jnp.float32)],
    )(part2, h_pad, wl2t, bl2r, wr2t, wot, bor)

    return (out_full[:n], h2_full[:n])


# speedup vs baseline: 7.6673x; 1.6866x over previous
import functools

import jax
import jax.numpy as jnp
from jax import lax
from jax.experimental import pallas as pl
from jax.experimental.pallas import tpu as pltpu
from jax.experimental.pallas import tpu_sc as plsc

NC = 2
NS = 16
NW = NC * NS
CH = 40


def _sc_aggregate(feats, src3, dst3, zrow, n_pad, dp, nch):
    rows_tile = n_pad // NS
    mesh = plsc.VectorSubcoreMesh(core_axis_name="c", subcore_axis_name="s")

    @functools.partial(
        pl.kernel,
        mesh=mesh,
        compiler_params=pltpu.CompilerParams(use_tc_tiling_on_sc=False),
        out_type=jax.ShapeDtypeStruct((NC, n_pad, dp), jnp.float32),
        scratch_types=[
            pltpu.VMEM((nch, CH), jnp.int32),
            pltpu.VMEM((nch, CH), jnp.int32),
            pltpu.VMEM((CH, dp), jnp.float32),
            pltpu.VMEM((CH, dp), jnp.float32),
            pltpu.VMEM_SHARED((n_pad, dp), jnp.float32),
            pltpu.SemaphoreType.DMA,
            pltpu.SemaphoreType.DMA,
            pltpu.SemaphoreType.DMA,
            pltpu.SemaphoreType.DMA,
        ],
    )
    def k(feats_hbm, src_hbm, dst_hbm, zrow_hbm, out_hbm,
          sidx, didx, rows0, rows1, acc, gs0, gs1, ss0, ss1):
        c = lax.axis_index("c")
        s = lax.axis_index("s")
        wid = s * NC + c

        pltpu.sync_copy(zrow_hbm, acc.at[pl.ds(s * rows_tile, rows_tile)])
        pltpu.sync_copy(src_hbm.at[wid], sidx)
        pltpu.sync_copy(dst_hbm.at[wid], didx)
        plsc.subcore_barrier()

        def g_desc(ch, buf, sem):
            return pltpu.make_async_copy(feats_hbm.at[sidx.at[ch]], buf, sem)

        def s_desc(ch, buf, sem):
            return pltpu.make_async_copy(buf, acc.at[didx.at[ch]], sem)

        def body(g, carry):
            ch0 = 2 * g

            @pl.when(g > 0)
            def _():
                s_desc(ch0 - 2, rows0, ss0).wait()

            g_desc(ch0, rows0, gs0).start()

            @pl.when(g > 0)
            def _():
                s_desc(ch0 - 1, rows1, ss1).wait()

            g_desc(ch0 + 1, rows1, gs1).start()
            g_desc(ch0, rows0, gs0).wait()
            s_desc(ch0, rows0, ss0).start(add=True)
            g_desc(ch0 + 1, rows1, gs1).wait()
            s_desc(ch0 + 1, rows1, ss1).start(add=True)
            return carry

        lax.fori_loop(0, nch // 2, body, 0)
        s_desc(nch - 2, rows0, ss0).wait()
        s_desc(nch - 1, rows1, ss1).wait()
        plsc.subcore_barrier()

        pltpu.sync_copy(acc.at[pl.ds(s * rows_tile, rows_tile)],
                        out_hbm.at[c, pl.ds(s * rows_tile, rows_tile)])

    return k(feats, src3, dst3, zrow)


def _combine1_body(p_ref, x_ref, wl_ref, bl_ref, wr_ref, o_ref, *, d):
    p = p_ref[0] + p_ref[1]
    cnt = jnp.maximum(p[:, d:d + 1], 1.0)
    mean = p[:, :d] / cnt
    y = jnp.dot(mean, wl_ref[...], preferred_element_type=jnp.float32)
    y += jnp.dot(x_ref[:, :d], wr_ref[...], preferred_element_type=jnp.float32)
    y += bl_ref[...]
    o_ref[...] = jnp.maximum(y, 0.0)


def _combine2_body(p_ref, c_ref, h_ref, wl_ref, bl_ref, wr_ref, wo_ref,
                   bo_ref, out_ref, h2_ref, *, d):
    p = p_ref[0] + p_ref[1]
    cnt = jnp.maximum(c_ref[0][:, d:d + 1] + c_ref[1][:, d:d + 1], 1.0)
    mean = p / cnt
    h2 = jnp.dot(mean, wl_ref[...], preferred_element_type=jnp.float32)
    h2 += jnp.dot(h_ref[...], wr_ref[...], preferred_element_type=jnp.float32)
    h2 += bl_ref[...]
    h2_ref[...] = h2
    out = jnp.dot(h2, wo_ref[...], preferred_element_type=jnp.float32)
    out_ref[...] = out + bo_ref[...]


def kernel(x, edge_index, Wl1, bl1, Wr1, Wl2, bl2, Wr2, Wo, bo):
    n, d = x.shape
    e = edge_index.shape[1]
    dp = d + 16
    n_pad = ((n + 8 * NW - 1) // (8 * NW)) * (8 * NW)
    e_pad = ((e + 2 * NW * CH - 1) // (2 * NW * CH)) * (2 * NW * CH)
    nch = e_pad // (NW * CH)
    rows_tile = n_pad // NS
    blk = 400
    grid = (n // blk,)

    src_flat, dst_flat = edge_index[0], edge_index[1]
    if e_pad != e:
        j = jnp.arange(e_pad - e, dtype=jnp.int32)
        src_flat = jnp.concatenate([src_flat, j % n])
        dst_flat = jnp.concatenate([dst_flat, n + j % (n_pad - n)])
    src3 = src_flat.reshape(NW, nch, CH)
    dst3 = dst_flat.reshape(NW, nch, CH)
    x_aug = jnp.concatenate(
        [x, jnp.ones((n, 1), jnp.float32), jnp.zeros((n, dp - d - 1), jnp.float32)],
        axis=1)
    z_dp = jnp.zeros((rows_tile, dp), jnp.float32)
    z_d = jnp.zeros((rows_tile, d), jnp.float32)
    wl1t, wr1t = Wl1.T, Wr1.T
    wl2t, wr2t, wot = Wl2.T, Wr2.T, Wo.T
    bl1r, bl2r, bor = bl1.reshape(1, d), bl2.reshape(1, d), bo.reshape(1, d)

    wspec = pl.BlockSpec((d, d), lambda i: (0, 0))
    bspec = pl.BlockSpec((1, d), lambda i: (0, 0))
    p1spec = pl.BlockSpec((NC, blk, dp), lambda i: (0, i, 0))
    p2spec = pl.BlockSpec((NC, blk, d), lambda i: (0, i, 0))
    cspec = pl.BlockSpec((NC, blk, dp), lambda i: (0, i, 0))
    fspec = pl.BlockSpec((blk, d), lambda i: (i, 0))

    part1 = _sc_aggregate(x_aug, src3, dst3, z_dp, n_pad, dp, nch)
    h = pl.pallas_call(
        functools.partial(_combine1_body, d=d),
        grid=grid,
        in_specs=[p1spec, fspec, wspec, bspec, wspec],
        out_specs=fspec,
        out_shape=jax.ShapeDtypeStruct((n, d), jnp.float32),
    )(part1, x_aug, wl1t, bl1r, wr1t)

    part2 = _sc_aggregate(h, src3, dst3, z_d, n_pad, d, nch)
    out, h2 = pl.pallas_call(
        functools.partial(_combine2_body, d=d),
        grid=grid,
        in_specs=[p2spec, cspec, fspec, wspec, bspec, wspec, wspec, bspec],
        out_specs=[fspec, fspec],
        out_shape=[jax.ShapeDtypeStruct((n, d), jnp.float32),
                   jax.ShapeDtypeStruct((n, d), jnp.float32)],
    )(part2, part1, h, wl2t, bl2r, wr2t, wot, bor)

    return (out, h2)


# speedup vs baseline: 8.2787x; 1.0797x over previous
import functools
import math

import jax
import jax.numpy as jnp
from jax import lax
from jax.experimental import pallas as pl
from jax.experimental.pallas import tpu as pltpu
from jax.experimental.pallas import tpu_sc as plsc

NC = 2
NS = 16
NW = NC * NS
CH1 = 48
CH2 = 80


def _sc_aggregate(feats, src3, dst3, zrow, n_pad, dp, nch, ch):
    rows_tile = n_pad // NS
    mesh = plsc.VectorSubcoreMesh(core_axis_name="c", subcore_axis_name="s")

    @functools.partial(
        pl.kernel,
        mesh=mesh,
        compiler_params=pltpu.CompilerParams(use_tc_tiling_on_sc=False),
        out_type=jax.ShapeDtypeStruct((NC, n_pad, dp), jnp.float32),
        scratch_types=[
            pltpu.VMEM((nch, ch), jnp.int32),
            pltpu.VMEM((nch, ch), jnp.int32),
            pltpu.VMEM((ch, dp), jnp.float32),
            pltpu.VMEM((ch, dp), jnp.float32),
            pltpu.VMEM_SHARED((n_pad, dp), jnp.float32),
            pltpu.SemaphoreType.DMA,
            pltpu.SemaphoreType.DMA,
            pltpu.SemaphoreType.DMA,
            pltpu.SemaphoreType.DMA,
        ],
    )
    def k(feats_hbm, src_hbm, dst_hbm, zrow_hbm, out_hbm,
          sidx, didx, rows0, rows1, acc, gs0, gs1, ss0, ss1):
        c = lax.axis_index("c")
        s = lax.axis_index("s")
        wid = s * NC + c

        pltpu.sync_copy(zrow_hbm, acc.at[pl.ds(s * rows_tile, rows_tile)])
        pltpu.sync_copy(src_hbm.at[wid], sidx)
        pltpu.sync_copy(dst_hbm.at[wid], didx)
        plsc.subcore_barrier()

        def g_desc(ch, buf, sem):
            return pltpu.make_async_copy(feats_hbm.at[sidx.at[ch]], buf, sem)

        def s_desc(ch, buf, sem):
            return pltpu.make_async_copy(buf, acc.at[didx.at[ch]], sem)

        def body(g, carry):
            ch0 = 2 * g

            @pl.when(g > 0)
            def _():
                s_desc(ch0 - 2, rows0, ss0).wait()

            g_desc(ch0, rows0, gs0).start()

            @pl.when(g > 0)
            def _():
                s_desc(ch0 - 1, rows1, ss1).wait()

            g_desc(ch0 + 1, rows1, gs1).start()
            g_desc(ch0, rows0, gs0).wait()
            s_desc(ch0, rows0, ss0).start(add=True)
            g_desc(ch0 + 1, rows1, gs1).wait()
            s_desc(ch0 + 1, rows1, ss1).start(add=True)
            return carry

        lax.fori_loop(0, nch // 2, body, 0)
        s_desc(nch - 2, rows0, ss0).wait()
        s_desc(nch - 1, rows1, ss1).wait()
        plsc.subcore_barrier()

        pltpu.sync_copy(acc.at[pl.ds(s * rows_tile, rows_tile)],
                        out_hbm.at[c, pl.ds(s * rows_tile, rows_tile)])

    return k(feats, src3, dst3, zrow)


def _combine1_body(p_ref, x_ref, wl_ref, bl_ref, wr_ref, o_ref, *, d):
    p = p_ref[0] + p_ref[1]
    cnt = jnp.maximum(p[:, d:d + 1], 1.0)
    mean = p[:, :d] / cnt
    y = jnp.dot(mean, wl_ref[...], preferred_element_type=jnp.float32)
    y += jnp.dot(x_ref[:, :d], wr_ref[...], preferred_element_type=jnp.float32)
    y += bl_ref[...]
    o_ref[...] = jnp.maximum(y, 0.0)


def _combine2_body(p_ref, c_ref, h_ref, wl_ref, bl_ref, wr_ref, wo_ref,
                   bo_ref, out_ref, h2_ref, *, d):
    p = p_ref[0] + p_ref[1]
    cnt = jnp.maximum(c_ref[0][:, d:d + 1] + c_ref[1][:, d:d + 1], 1.0)
    mean = p / cnt
    h2 = jnp.dot(mean, wl_ref[...], preferred_element_type=jnp.float32)
    h2 += jnp.dot(h_ref[...], wr_ref[...], preferred_element_type=jnp.float32)
    h2 += bl_ref[...]
    h2_ref[...] = h2
    out = jnp.dot(h2, wo_ref[...], preferred_element_type=jnp.float32)
    out_ref[...] = out + bo_ref[...]


def kernel(x, edge_index, Wl1, bl1, Wr1, Wl2, bl2, Wr2, Wo, bo):
    n, d = x.shape
    e = edge_index.shape[1]
    dp = d + 16
    n_pad = ((n + 8 * NW - 1) // (8 * NW)) * (8 * NW)
    unit = NW * (2 * CH1 * CH2 // math.gcd(CH1, CH2))
    e_pad = ((e + unit - 1) // unit) * unit
    nch1 = e_pad // (NW * CH1)
    nch2 = e_pad // (NW * CH2)
    rows_tile = n_pad // NS
    blk = 400
    grid = (n // blk,)

    src_flat, dst_flat = edge_index[0], edge_index[1]
    if e_pad != e:
        j = jnp.arange(e_pad - e, dtype=jnp.int32)
        src_flat = jnp.concatenate([src_flat, j % n])
        dst_flat = jnp.concatenate([dst_flat, n + j % (n_pad - n)])
    src3a = src_flat.reshape(NW, nch1, CH1)
    dst3a = dst_flat.reshape(NW, nch1, CH1)
    src3b = src_flat.reshape(NW, nch2, CH2)
    dst3b = dst_flat.reshape(NW, nch2, CH2)
    x_aug = jnp.concatenate(
        [x, jnp.ones((n, 1), jnp.float32), jnp.zeros((n, dp - d - 1), jnp.float32)],
        axis=1)
    z_dp = jnp.zeros((rows_tile, dp), jnp.float32)
    z_d = jnp.zeros((rows_tile, d), jnp.float32)
    wl1t, wr1t = Wl1.T, Wr1.T
    wl2t, wr2t, wot = Wl2.T, Wr2.T, Wo.T
    bl1r, bl2r, bor = bl1.reshape(1, d), bl2.reshape(1, d), bo.reshape(1, d)

    wspec = pl.BlockSpec((d, d), lambda i: (0, 0))
    bspec = pl.BlockSpec((1, d), lambda i: (0, 0))
    p1spec = pl.BlockSpec((NC, blk, dp), lambda i: (0, i, 0))
    p2spec = pl.BlockSpec((NC, blk, d), lambda i: (0, i, 0))
    cspec = pl.BlockSpec((NC, blk, dp), lambda i: (0, i, 0))
    fspec = pl.BlockSpec((blk, d), lambda i: (i, 0))

    part1 = _sc_aggregate(x_aug, src3a, dst3a, z_dp, n_pad, dp, nch1, CH1)
    h = pl.pallas_call(
        functools.partial(_combine1_body, d=d),
        grid=grid,
        in_specs=[p1spec, fspec, wspec, bspec, wspec],
        out_specs=fspec,
        out_shape=jax.ShapeDtypeStruct((n, d), jnp.float32),
    )(part1, x_aug, wl1t, bl1r, wr1t)

    part2 = _sc_aggregate(h, src3b, dst3b, z_d, n_pad, d, nch2, CH2)
    out, h2 = pl.pallas_call(
        functools.partial(_combine2_body, d=d),
        grid=grid,
        in_specs=[p2spec, cspec, fspec, wspec, bspec, wspec, wspec, bspec],
        out_specs=[fspec, fspec],
        out_shape=[jax.ShapeDtypeStruct((n, d), jnp.float32),
                   jax.ShapeDtypeStruct((n, d), jnp.float32)],
    )(part2, part1, h, wl2t, bl2r, wr2t, wot, bor)

    return (out, h2)


# speedup vs baseline: 9.0695x; 1.0955x over previous
import functools

import jax
import jax.numpy as jnp
from jax import lax
from jax.experimental import pallas as pl
from jax.experimental.pallas import tpu as pltpu
from jax.experimental.pallas import tpu_sc as plsc

NC = 2
NS = 16
NW = NC * NS
CH1, NBUF1 = 48, 2
CH2, NBUF2 = 56, 4


def _sc_aggregate(feats, src3, dst3, zrow, n_pad, dp, nch, ch, nbuf):
    rows_tile = n_pad // NS
    mesh = plsc.VectorSubcoreMesh(core_axis_name="c", subcore_axis_name="s")

    @functools.partial(
        pl.kernel,
        mesh=mesh,
        compiler_params=pltpu.CompilerParams(use_tc_tiling_on_sc=False),
        out_type=jax.ShapeDtypeStruct((NC, n_pad, dp), jnp.float32),
        scratch_types=(
            [pltpu.VMEM((nch, ch), jnp.int32)] * 2
            + [pltpu.VMEM((ch, dp), jnp.float32)] * nbuf
            + [pltpu.VMEM_SHARED((n_pad, dp), jnp.float32)]
            + [pltpu.SemaphoreType.DMA] * (2 * nbuf)
        ),
    )
    def k(feats_hbm, src_hbm, dst_hbm, zrow_hbm, out_hbm, sidx, didx, *rest):
        rows = rest[:nbuf]
        acc = rest[nbuf]
        gsem = rest[nbuf + 1:2 * nbuf + 1]
        ssem = rest[2 * nbuf + 1:]
        c = lax.axis_index("c")
        s = lax.axis_index("s")
        wid = s * NC + c

        pltpu.sync_copy(zrow_hbm, acc.at[pl.ds(s * rows_tile, rows_tile)])
        pltpu.sync_copy(src_hbm.at[wid], sidx)
        pltpu.sync_copy(dst_hbm.at[wid], didx)
        plsc.subcore_barrier()

        def g_desc(chk, buf, sem):
            return pltpu.make_async_copy(feats_hbm.at[sidx.at[chk]], buf, sem)

        def s_desc(chk, buf, sem):
            return pltpu.make_async_copy(buf, acc.at[didx.at[chk]], sem)

        def body(g, carry):
            ch0 = nbuf * g
            for b in range(nbuf):
                @pl.when(g > 0)
                def _(b=b):
                    s_desc(ch0 - nbuf + b, rows[b], ssem[b]).wait()

                g_desc(ch0 + b, rows[b], gsem[b]).start()
            for b in range(nbuf):
                g_desc(ch0 + b, rows[b], gsem[b]).wait()
                s_desc(ch0 + b, rows[b], ssem[b]).start(add=True)
            return carry

        lax.fori_loop(0, nch // nbuf, body, 0)
        for b in range(nbuf):
            s_desc(nch - nbuf + b, rows[b], ssem[b]).wait()
        plsc.subcore_barrier()

        pltpu.sync_copy(acc.at[pl.ds(s * rows_tile, rows_tile)],
                        out_hbm.at[c, pl.ds(s * rows_tile, rows_tile)])

    return k(feats, src3, dst3, zrow)


def _edges(src_flat, dst_flat, e, n, n_pad, ch, nbuf):
    unit = NW * ch * nbuf
    e_pad = ((e + unit - 1) // unit) * unit
    nch = e_pad // (NW * ch)
    if e_pad != e:
        j = jnp.arange(e_pad - e, dtype=jnp.int32)
        src_flat = jnp.concatenate([src_flat, j % n])
        dst_flat = jnp.concatenate([dst_flat, n + j % (n_pad - n)])
    return src_flat.reshape(NW, nch, ch), dst_flat.reshape(NW, nch, ch), nch


def _combine1_body(p_ref, x_ref, wl_ref, bl_ref, wr_ref, o_ref, *, d):
    p = p_ref[0] + p_ref[1]
    cnt = jnp.maximum(p[:, d:d + 1], 1.0)
    mean = p[:, :d] / cnt
    y = jnp.dot(mean, wl_ref[...], preferred_element_type=jnp.float32)
    y += jnp.dot(x_ref[:, :d], wr_ref[...], preferred_element_type=jnp.float32)
    y += bl_ref[...]
    o_ref[...] = jnp.maximum(y, 0.0)


def _combine2_body(p_ref, c_ref, h_ref, wl_ref, bl_ref, wr_ref, wo_ref,
                   bo_ref, out_ref, h2_ref, *, d):
    p = p_ref[0] + p_ref[1]
    cnt = jnp.maximum(c_ref[0][:, d:d + 1] + c_ref[1][:, d:d + 1], 1.0)
    mean = p / cnt
    h2 = jnp.dot(mean, wl_ref[...], preferred_element_type=jnp.float32)
    h2 += jnp.dot(h_ref[...], wr_ref[...], preferred_element_type=jnp.float32)
    h2 += bl_ref[...]
    h2_ref[...] = h2
    out = jnp.dot(h2, wo_ref[...], preferred_element_type=jnp.float32)
    out_ref[...] = out + bo_ref[...]


def kernel(x, edge_index, Wl1, bl1, Wr1, Wl2, bl2, Wr2, Wo, bo):
    n, d = x.shape
    e = edge_index.shape[1]
    dp = d + 16
    n_pad = ((n + 8 * NS - 1) // (8 * NS)) * (8 * NS)
    rows_tile = n_pad // NS
    blk = 400
    grid = (n // blk,)

    src3a, dst3a, nch1 = _edges(edge_index[0], edge_index[1], e, n, n_pad,
                                CH1, NBUF1)
    src3b, dst3b, nch2 = _edges(edge_index[0], edge_index[1], e, n, n_pad,
                                CH2, NBUF2)
    x_aug = jnp.concatenate(
        [x, jnp.ones((n, 1), jnp.float32), jnp.zeros((n, dp - d - 1), jnp.float32)],
        axis=1)
    z_dp = jnp.zeros((rows_tile, dp), jnp.float32)
    z_d = jnp.zeros((rows_tile, d), jnp.float32)
    wl1t, wr1t = Wl1.T, Wr1.T
    wl2t, wr2t, wot = Wl2.T, Wr2.T, Wo.T
    bl1r, bl2r, bor = bl1.reshape(1, d), bl2.reshape(1, d), bo.reshape(1, d)

    wspec = pl.BlockSpec((d, d), lambda i: (0, 0))
    bspec = pl.BlockSpec((1, d), lambda i: (0, 0))
    p1spec = pl.BlockSpec((NC, blk, dp), lambda i: (0, i, 0))
    p2spec = pl.BlockSpec((NC, blk, d), lambda i: (0, i, 0))
    fspec = pl.BlockSpec((blk, d), lambda i: (i, 0))

    part1 = _sc_aggregate(x_aug, src3a, dst3a, z_dp, n_pad, dp, nch1, CH1, NBUF1)
    h = pl.pallas_call(
        functools.partial(_combine1_body, d=d),
        grid=grid,
        in_specs=[p1spec, fspec, wspec, bspec, wspec],
        out_specs=fspec,
        out_shape=jax.ShapeDtypeStruct((n, d), jnp.float32),
    )(part1, x_aug, wl1t, bl1r, wr1t)

    part2 = _sc_aggregate(h, src3b, dst3b, z_d, n_pad, d, nch2, CH2, NBUF2)
    out, h2 = pl.pallas_call(
        functools.partial(_combine2_body, d=d),
        grid=grid,
        in_specs=[p2spec, p1spec, fspec, wspec, bspec, wspec, wspec, bspec],
        out_specs=[fspec, fspec],
        out_shape=[jax.ShapeDtypeStruct((n, d), jnp.float32),
                   jax.ShapeDtypeStruct((n, d), jnp.float32)],
    )(part2, part1, h, wl2t, bl2r, wr2t, wot, bor)

    return (out, h2)


# speedup vs baseline: 10.3660x; 1.1429x over previous
import functools

import jax
import jax.numpy as jnp
from jax import lax
from jax.experimental import pallas as pl
from jax.experimental.pallas import tpu as pltpu
from jax.experimental.pallas import tpu_sc as plsc

NC = 2
NS = 16
NW = NC * NS
CH1, NBUF1 = 32, 4
CH2, NBUF2 = 56, 4


def _sc_aggregate(feats, src3, dst3, zrow, n_pad, dp, nch, ch, nbuf):
    rows_tile = n_pad // NS
    mesh = plsc.VectorSubcoreMesh(core_axis_name="c", subcore_axis_name="s")

    @functools.partial(
        pl.kernel,
        mesh=mesh,
        compiler_params=pltpu.CompilerParams(use_tc_tiling_on_sc=False),
        out_type=jax.ShapeDtypeStruct((NC, n_pad, dp), jnp.float32),
        scratch_types=(
            [pltpu.VMEM((nch, ch), jnp.int32)] * 2
            + [pltpu.VMEM((ch, dp), jnp.float32)] * nbuf
            + [pltpu.VMEM_SHARED((n_pad, dp), jnp.float32)]
            + [pltpu.SemaphoreType.DMA] * (2 * nbuf)
        ),
    )
    def k(feats_hbm, src_hbm, dst_hbm, zrow_hbm, out_hbm, sidx, didx, *rest):
        rows = rest[:nbuf]
        acc = rest[nbuf]
        gsem = rest[nbuf + 1:2 * nbuf + 1]
        ssem = rest[2 * nbuf + 1:]
        c = lax.axis_index("c")
        s = lax.axis_index("s")
        wid = s * NC + c

        pltpu.sync_copy(zrow_hbm, acc.at[pl.ds(s * rows_tile, rows_tile)])
        pltpu.sync_copy(src_hbm.at[wid], sidx)
        pltpu.sync_copy(dst_hbm.at[wid], didx)
        plsc.subcore_barrier()

        def g_desc(chk, buf, sem):
            return pltpu.make_async_copy(feats_hbm.at[sidx.at[chk]], buf, sem)

        def s_desc(chk, buf, sem):
            return pltpu.make_async_copy(buf, acc.at[didx.at[chk]], sem)

        def body(g, carry):
            ch0 = nbuf * g
            for b in range(nbuf):
                @pl.when(g > 0)
                def _(b=b):
                    s_desc(ch0 - nbuf + b, rows[b], ssem[b]).wait()

                g_desc(ch0 + b, rows[b], gsem[b]).start()
            for b in range(nbuf):
                g_desc(ch0 + b, rows[b], gsem[b]).wait()
                s_desc(ch0 + b, rows[b], ssem[b]).start(add=True)
            return carry

        lax.fori_loop(0, nch // nbuf, body, 0)
        for b in range(nbuf):
            s_desc(nch - nbuf + b, rows[b], ssem[b]).wait()
        plsc.subcore_barrier()

        pltpu.sync_copy(acc.at[pl.ds(s * rows_tile, rows_tile)],
                        out_hbm.at[c, pl.ds(s * rows_tile, rows_tile)])

    return k(feats, src3, dst3, zrow)


def _edges(src_flat, dst_flat, e, n, n_pad, ch, nbuf):
    unit = NW * ch * nbuf
    e_pad = ((e + unit - 1) // unit) * unit
    nch = e_pad // (NW * ch)
    if e_pad != e:
        j = jnp.arange(e_pad - e, dtype=jnp.int32)
        src_flat = jnp.concatenate([src_flat, j % n])
        dst_flat = jnp.concatenate([dst_flat, n + j % (n_pad - n)])
    return src_flat.reshape(NW, nch, ch), dst_flat.reshape(NW, nch, ch), nch


def _combine1_body(p_ref, x_ref, wl_ref, bl_ref, wr_ref, o_ref, *, d):
    p = p_ref[0] + p_ref[1]
    cnt = jnp.maximum(p[:, d:d + 1], 1.0)
    mean = p[:, :d] / cnt
    y = jnp.dot(mean, wl_ref[...], preferred_element_type=jnp.float32)
    y += jnp.dot(x_ref[:, :d], wr_ref[...], preferred_element_type=jnp.float32)
    y += bl_ref[...]
    o_ref[...] = jnp.maximum(y, 0.0)


def _combine2_body(p_ref, c_ref, h_ref, wl_ref, bl_ref, wr_ref, wo_ref,
                   bo_ref, out_ref, h2_ref, *, d):
    p = p_ref[0] + p_ref[1]
    cnt = jnp.maximum(c_ref[0][:, d:d + 1] + c_ref[1][:, d:d + 1], 1.0)
    mean = p / cnt
    h2 = jnp.dot(mean, wl_ref[...], preferred_element_type=jnp.float32)
    h2 += jnp.dot(h_ref[...], wr_ref[...], preferred_element_type=jnp.float32)
    h2 += bl_ref[...]
    h2_ref[...] = h2
    out = jnp.dot(h2, wo_ref[...], preferred_element_type=jnp.float32)
    out_ref[...] = out + bo_ref[...]


def kernel(x, edge_index, Wl1, bl1, Wr1, Wl2, bl2, Wr2, Wo, bo):
    n, d = x.shape
    e = edge_index.shape[1]
    dp = d + 16
    n_pad = ((n + 8 * NS - 1) // (8 * NS)) * (8 * NS)
    rows_tile = n_pad // NS
    blk = 400
    grid = (n // blk,)

    src3a, dst3a, nch1 = _edges(edge_index[0], edge_index[1], e, n, n_pad,
                                CH1, NBUF1)
    src3b, dst3b, nch2 = _edges(edge_index[0], edge_index[1], e, n, n_pad,
                                CH2, NBUF2)
    x_aug = jnp.concatenate(
        [x, jnp.ones((n, 1), jnp.float32), jnp.zeros((n, dp - d - 1), jnp.float32)],
        axis=1)
    z_dp = jnp.zeros((rows_tile, dp), jnp.float32)
    z_d = jnp.zeros((rows_tile, d), jnp.float32)
    wl1t, wr1t = Wl1.T, Wr1.T
    wl2t, wr2t, wot = Wl2.T, Wr2.T, Wo.T
    bl1r, bl2r, bor = bl1.reshape(1, d), bl2.reshape(1, d), bo.reshape(1, d)

    wspec = pl.BlockSpec((d, d), lambda i: (0, 0))
    bspec = pl.BlockSpec((1, d), lambda i: (0, 0))
    p1spec = pl.BlockSpec((NC, blk, dp), lambda i: (0, i, 0))
    p2spec = pl.BlockSpec((NC, blk, d), lambda i: (0, i, 0))
    fspec = pl.BlockSpec((blk, d), lambda i: (i, 0))

    part1 = _sc_aggregate(x_aug, src3a, dst3a, z_dp, n_pad, dp, nch1, CH1, NBUF1)
    h = pl.pallas_call(
        functools.partial(_combine1_body, d=d),
        grid=grid,
        in_specs=[p1spec, fspec, wspec, bspec, wspec],
        out_specs=fspec,
        out_shape=jax.ShapeDtypeStruct((n, d), jnp.float32),
    )(part1, x_aug, wl1t, bl1r, wr1t)

    part2 = _sc_aggregate(h, src3b, dst3b, z_d, n_pad, d, nch2, CH2, NBUF2)
    out, h2 = pl.pallas_call(
        functools.partial(_combine2_body, d=d),
        grid=grid,
        in_specs=[p2spec, p1spec, fspec, wspec, bspec, wspec, wspec, bspec],
        out_specs=[fspec, fspec],
        out_shape=[jax.ShapeDtypeStruct((n, d), jnp.float32),
                   jax.ShapeDtypeStruct((n, d), jnp.float32)],
    )(part2, part1, h, wl2t, bl2r, wr2t, wot, bor)

    return (out, h2)


# speedup vs baseline: 10.7617x; 1.0382x over previous
import functools

import jax
import jax.numpy as jnp
from jax import lax
from jax.experimental import pallas as pl
from jax.experimental.pallas import tpu as pltpu
from jax.experimental.pallas import tpu_sc as plsc

NC = 2
NS = 16
NW = NC * NS
L = 16
CH1, NBUF1 = 32, 6
CH2, NBUF2 = 48, 6


def _sc_aggregate(feats, packed, zrow, n_pad, dp, nch, ch, nbuf):
    rows_tile = n_pad // NS
    mesh = plsc.VectorSubcoreMesh(core_axis_name="c", subcore_axis_name="s")

    @functools.partial(
        pl.kernel,
        mesh=mesh,
        compiler_params=pltpu.CompilerParams(use_tc_tiling_on_sc=False),
        out_type=jax.ShapeDtypeStruct((NC, n_pad, dp), jnp.float32),
        scratch_types=(
            [pltpu.VMEM((nch, ch), jnp.int32)]
            + [pltpu.VMEM((ch,), jnp.int32)] * (2 * nbuf)
            + [pltpu.VMEM((ch, dp), jnp.float32)] * nbuf
            + [pltpu.VMEM_SHARED((n_pad, dp), jnp.float32)]
            + [pltpu.SemaphoreType.DMA] * (2 * nbuf)
        ),
    )
    def k(feats_hbm, pk_hbm, zrow_hbm, out_hbm, pk, *rest):
        sidx = rest[:nbuf]
        didx = rest[nbuf:2 * nbuf]
        rows = rest[2 * nbuf:3 * nbuf]
        acc = rest[3 * nbuf]
        gsem = rest[3 * nbuf + 1:4 * nbuf + 1]
        ssem = rest[4 * nbuf + 1:]
        c = lax.axis_index("c")
        s = lax.axis_index("s")
        wid = s * NC + c

        pltpu.sync_copy(zrow_hbm, acc.at[pl.ds(s * rows_tile, rows_tile)])
        pltpu.sync_copy(pk_hbm.at[wid], pk)
        plsc.subcore_barrier()

        def unpack(chk, b):
            for j in range(ch // L):
                v = pk[chk, pl.ds(j * L, L)]
                sidx[b][pl.ds(j * L, L)] = lax.bitwise_and(v, 0xFFFF)
                didx[b][pl.ds(j * L, L)] = lax.shift_right_logical(v, 16)

        def g_desc(b, sem):
            return pltpu.make_async_copy(feats_hbm.at[sidx[b]], rows[b], sem)

        def s_desc(b, sem):
            return pltpu.make_async_copy(rows[b], acc.at[didx[b]], sem)

        def body(g, carry):
            ch0 = nbuf * g
            for b in range(nbuf):
                @pl.when(g > 0)
                def _(b=b):
                    s_desc(b, ssem[b]).wait()

                unpack(ch0 + b, b)
                g_desc(b, gsem[b]).start()
            for b in range(nbuf):
                g_desc(b, gsem[b]).wait()
                s_desc(b, ssem[b]).start(add=True)
            return carry

        lax.fori_loop(0, nch // nbuf, body, 0)
        for b in range(nbuf):
            s_desc(b, ssem[b]).wait()
        plsc.subcore_barrier()

        pltpu.sync_copy(acc.at[pl.ds(s * rows_tile, rows_tile)],
                        out_hbm.at[c, pl.ds(s * rows_tile, rows_tile)])

    return k(feats, packed, zrow)


def _edges(edge_index, e, n, n_pad, ch, nbuf):
    unit = NW * ch * nbuf
    e_pad = ((e + unit - 1) // unit) * unit
    nch = e_pad // (NW * ch)
    src_flat, dst_flat = edge_index[0], edge_index[1]
    if e_pad != e:
        j = jnp.arange(e_pad - e, dtype=jnp.int32)
        src_flat = jnp.concatenate([src_flat, j % n])
        dst_flat = jnp.concatenate([dst_flat, n + j % (n_pad - n)])
    packed = src_flat + dst_flat * 65536
    return packed.reshape(NW, nch, ch), nch


def _combine1_body(p_ref, x_ref, wl_ref, bl_ref, wr_ref, o_ref, *, d):
    p = p_ref[0] + p_ref[1]
    cnt = jnp.maximum(p[:, d:d + 1], 1.0)
    mean = p[:, :d] / cnt
    y = jnp.dot(mean, wl_ref[...], preferred_element_type=jnp.float32)
    y += jnp.dot(x_ref[:, :d], wr_ref[...], preferred_element_type=jnp.float32)
    y += bl_ref[...]
    o_ref[...] = jnp.maximum(y, 0.0)


def _combine2_body(p_ref, c_ref, h_ref, wl_ref, bl_ref, wr_ref, wo_ref,
                   bo_ref, out_ref, h2_ref, *, d):
    p = p_ref[0] + p_ref[1]
    cnt = jnp.maximum(c_ref[0][:, d:d + 1] + c_ref[1][:, d:d + 1], 1.0)
    mean = p / cnt
    h2 = jnp.dot(mean, wl_ref[...], preferred_element_type=jnp.float32)
    h2 += jnp.dot(h_ref[...], wr_ref[...], preferred_element_type=jnp.float32)
    h2 += bl_ref[...]
    h2_ref[...] = h2
    out = jnp.dot(h2, wo_ref[...], preferred_element_type=jnp.float32)
    out_ref[...] = out + bo_ref[...]


def kernel(x, edge_index, Wl1, bl1, Wr1, Wl2, bl2, Wr2, Wo, bo):
    n, d = x.shape
    e = edge_index.shape[1]
    dp = d + 16
    n_pad = ((n + 8 * NS - 1) // (8 * NS)) * (8 * NS)
    rows_tile = n_pad // NS
    blk = 400
    grid = (n // blk,)

    pk1, nch1 = _edges(edge_index, e, n, n_pad, CH1, NBUF1)
    pk2, nch2 = _edges(edge_index, e, n, n_pad, CH2, NBUF2)
    x_aug = jnp.concatenate(
        [x, jnp.ones((n, 1), jnp.float32), jnp.zeros((n, dp - d - 1), jnp.float32)],
        axis=1)
    z_dp = jnp.zeros((rows_tile, dp), jnp.float32)
    z_d = jnp.zeros((rows_tile, d), jnp.float32)
    wl1t, wr1t = Wl1.T, Wr1.T
    wl2t, wr2t, wot = Wl2.T, Wr2.T, Wo.T
    bl1r, bl2r, bor = bl1.reshape(1, d), bl2.reshape(1, d), bo.reshape(1, d)

    wspec = pl.BlockSpec((d, d), lambda i: (0, 0))
    bspec = pl.BlockSpec((1, d), lambda i: (0, 0))
    p1spec = pl.BlockSpec((NC, blk, dp), lambda i: (0, i, 0))
    p2spec = pl.BlockSpec((NC, blk, d), lambda i: (0, i, 0))
    fspec = pl.BlockSpec((blk, d), lambda i: (i, 0))

    part1 = _sc_aggregate(x_aug, pk1, z_dp, n_pad, dp, nch1, CH1, NBUF1)
    h = pl.pallas_call(
        functools.partial(_combine1_body, d=d),
        grid=grid,
        in_specs=[p1spec, fspec, wspec, bspec, wspec],
        out_specs=fspec,
        out_shape=jax.ShapeDtypeStruct((n, d), jnp.float32),
    )(part1, x_aug, wl1t, bl1r, wr1t)

    part2 = _sc_aggregate(h, pk2, z_d, n_pad, d, nch2, CH2, NBUF2)
    out, h2 = pl.pallas_call(
        functools.partial(_combine2_body, d=d),
        grid=grid,
        in_specs=[p2spec, p1spec, fspec, wspec, bspec, wspec, wspec, bspec],
        out_specs=[fspec, fspec],
        out_shape=[jax.ShapeDtypeStruct((n, d), jnp.float32),
                   jax.ShapeDtypeStruct((n, d), jnp.float32)],
    )(part2, part1, h, wl2t, bl2r, wr2t, wot, bor)

    return (out, h2)
